# Initial kernel scaffold; baseline (speedup 1.0000x reference)
import functools

import jax
import jax.numpy as jnp
from jax import lax
from jax.experimental import pallas as pl
from jax.experimental.pallas import tpu as pltpu
from jax.experimental.pallas import tpu_sc as plsc

N = 10000
E = 320000
D = 128
DE = 4
H = 128
OUT = 10
G = 16

NC = 2
NS = 16
NW = NC * NS

C = 128
KCH = 79
EW = KCH * C
EPAD = NW * EW
N8 = N + 8
RPT = N // NS
RCH = 125

RB = 1000
NB = N // RB


def _sc_body(with_ea, *refs):
    if with_ea:
        (table, src3, dst3, ea3, zer_e, out_x, out_e,
         src_idx, dst_idx, rows, ea_st, sem, acc_x, acc_e) = refs
    else:
        (table, src3, dst3, out_x,
         src_idx, dst_idx, rows, sem, acc_x) = refs
    c = lax.axis_index("c")
    s = lax.axis_index("s")
    wid = s * NC + c

    pltpu.sync_copy(src3.at[wid], src_idx)
    pltpu.sync_copy(dst3.at[wid], dst_idx)

    base = s * RPT
    for k in range(RPT // RCH):
        off = base + k * RCH
        pltpu.sync_copy(table.at[pl.ds(off, RCH)], rows.at[pl.ds(0, RCH)])
        pltpu.sync_copy(rows.at[pl.ds(0, RCH)], acc_x.at[pl.ds(off, RCH)])
        if with_ea:
            pltpu.sync_copy(zer_e.at[pl.ds(off, RCH)], ea_st.at[pl.ds(0, RCH)])
            pltpu.sync_copy(ea_st.at[pl.ds(0, RCH)], acc_e.at[pl.ds(off, RCH)])
    plsc.subcore_barrier()

    def chunk(j, carry):
        pltpu.async_copy(table.at[src_idx.at[j]], rows, sem).wait()
        pltpu.sync_copy(rows, acc_x.at[dst_idx.at[j]], add=True)
        if with_ea:
            pltpu.sync_copy(ea3.at[wid, j], ea_st)
            pltpu.sync_copy(ea_st, acc_e.at[dst_idx.at[j]], add=True)
        return carry

    lax.fori_loop(0, KCH, chunk, 0)
    plsc.subcore_barrier()

    for k in range(RPT // RCH):
        off = base + k * RCH
        pltpu.sync_copy(acc_x.at[pl.ds(off, RCH)], rows.at[pl.ds(0, RCH)])
        pltpu.sync_copy(rows.at[pl.ds(0, RCH)], out_x.at[c, pl.ds(off, RCH)])
        if with_ea:
            pltpu.sync_copy(acc_e.at[pl.ds(off, RCH)], ea_st.at[pl.ds(0, RCH)])
            pltpu.sync_copy(ea_st.at[pl.ds(0, RCH)], out_e.at[c, pl.ds(off, RCH)])


def _make_sc_scatter(with_ea):
    mesh = plsc.VectorSubcoreMesh(core_axis_name="c", subcore_axis_name="s")
    out_type = [jax.ShapeDtypeStruct((NC, N8, D), jnp.float32)]
    scratch = [
        pltpu.VMEM((KCH, C), jnp.int32),
        pltpu.VMEM((KCH, C), jnp.int32),
        pltpu.VMEM((C, D), jnp.float32),
    ]
    if with_ea:
        out_type.append(jax.ShapeDtypeStruct((NC, N8, DE), jnp.float32))
        scratch.append(pltpu.VMEM((C, DE), jnp.float32))
    scratch.append(pltpu.SemaphoreType.DMA)
    scratch.append(pltpu.VMEM_SHARED((N8, D), jnp.float32))
    if with_ea:
        scratch.append(pltpu.VMEM_SHARED((N8, DE), jnp.float32))
    return pl.kernel(
        functools.partial(_sc_body, with_ea),
        out_type=tuple(out_type) if with_ea else out_type[0],
        mesh=mesh,
        scratch_types=scratch,
    )


def _tc_layer_body(S_ref, x_ref, ae_ref, Wx_ref, We_ref, b_ref, o_ref):
    aggr = S_ref[0] + S_ref[1] - x_ref[...]
    ae = ae_ref[0] + ae_ref[1]
    acc = jnp.dot(aggr, Wx_ref[...], preferred_element_type=jnp.float32)
    acc = acc + jnp.dot(ae, We_ref[...], preferred_element_type=jnp.float32)
    acc = acc + b_ref[...]
    o_ref[...] = jnp.maximum(acc, 0.0)


def _tc_final_body(S_ref, h1_ref, ae_ref, Wx_ref, We_ref, b_ref, batch_ref,
                   Wout_ref, bout_ref, o_ref, pool_acc, cnt_acc):
    i = pl.program_id(0)

    @pl.when(i == 0)
    def _init():
        pool_acc[...] = jnp.zeros_like(pool_acc)
        cnt_acc[...] = jnp.zeros_like(cnt_acc)

    aggr = S_ref[0] + S_ref[1] - h1_ref[...]
    ae = ae_ref[0] + ae_ref[1]
    h = jnp.dot(aggr, Wx_ref[...], preferred_element_type=jnp.float32)
    h = h + jnp.dot(ae, We_ref[...], preferred_element_type=jnp.float32)
    h = jnp.maximum(h + b_ref[...], 0.0)

    b_row = batch_ref[0]
    gids = lax.broadcasted_iota(jnp.int32, (G, 1), 0)
    onehot = (b_row == gids).astype(jnp.float32)
    pool_acc[...] += lax.dot_general(
        onehot, h, (((1,), (0,)), ((), ())),
        preferred_element_type=jnp.float32)
    cnt_acc[...] += lax.dot_general(
        onehot, jnp.ones((RB, H), jnp.float32), (((1,), (0,)), ((), ())),
        preferred_element_type=jnp.float32)

    @pl.when(i == NB - 1)
    def _final():
        pooled = pool_acc[...] / jnp.maximum(cnt_acc[...], 1.0)
        logits = jnp.dot(pooled, Wout_ref[...],
                         preferred_element_type=jnp.float32) + bout_ref[...]
        m = jnp.max(logits, axis=1, keepdims=True)
        z = logits - m
        lse = jnp.log(jnp.sum(jnp.exp(z), axis=1, keepdims=True))
        o_ref[...] = z - lse


_sc_scatter_ea = _make_sc_scatter(True)
_sc_scatter = _make_sc_scatter(False)

_tc_layer1 = pl.pallas_call(
    _tc_layer_body,
    grid=(NB,),
    in_specs=[
        pl.BlockSpec((NC, RB, D), lambda i: (0, i, 0)),
        pl.BlockSpec((RB, D), lambda i: (i, 0)),
        pl.BlockSpec((NC, RB, DE), lambda i: (0, i, 0)),
        pl.BlockSpec((D, H), lambda i: (0, 0)),
        pl.BlockSpec((DE, H), lambda i: (0, 0)),
        pl.BlockSpec((1, H), lambda i: (0, 0)),
    ],
    out_specs=pl.BlockSpec((RB, H), lambda i: (i, 0)),
    out_shape=jax.ShapeDtypeStruct((N, H), jnp.float32),
)

_tc_final = pl.pallas_call(
    _tc_final_body,
    grid=(NB,),
    in_specs=[
        pl.BlockSpec((NC, RB, H), lambda i: (0, i, 0)),
        pl.BlockSpec((RB, H), lambda i: (i, 0)),
        pl.BlockSpec((NC, RB, DE), lambda i: (0, i, 0)),
        pl.BlockSpec((H, H), lambda i: (0, 0)),
        pl.BlockSpec((DE, H), lambda i: (0, 0)),
        pl.BlockSpec((1, H), lambda i: (0, 0)),
        pl.BlockSpec((1, 1, RB), lambda i: (i, 0, 0)),
        pl.BlockSpec((H, OUT), lambda i: (0, 0)),
        pl.BlockSpec((1, OUT), lambda i: (0, 0)),
    ],
    out_specs=pl.BlockSpec((G, OUT), lambda i: (0, 0)),
    out_shape=jax.ShapeDtypeStruct((G, OUT), jnp.float32),
    scratch_shapes=[
        pltpu.VMEM((G, H), jnp.float32),
        pltpu.VMEM((G, H), jnp.float32),
    ],
)


def kernel(x, edge_index, edge_attr, batch, W0, b0, W1, b1, Wout, bout):
    pad = EPAD - E
    src = jnp.concatenate([edge_index[0], jnp.zeros((pad,), edge_index.dtype)])
    dst = jnp.concatenate([edge_index[1], jnp.full((pad,), N, edge_index.dtype)])
    srcp = src.reshape(NW, KCH, C)
    dstp = dst.reshape(NW, KCH, C)
    eap = jnp.concatenate(
        [edge_attr, jnp.zeros((pad, DE), edge_attr.dtype)]).reshape(NW, KCH, C, DE)
    zer_e = jnp.zeros((N8, DE), jnp.float32)

    S1, Ae = _sc_scatter_ea(x, srcp, dstp, eap, zer_e)
    h1 = _tc_layer1(S1, x, Ae, W0[:D], W0[D:], b0.reshape(1, H))
    S2 = _sc_scatter(h1, srcp, dstp)
    return _tc_final(S2, h1, Ae, W1[:H], W1[H:], b1.reshape(1, H),
                     batch.reshape(NB, 1, RB), Wout, bout.reshape(1, OUT))


# speedup vs baseline: 3.2605x; 3.2605x over previous
import jax
import jax.numpy as jnp
from jax import lax
from jax.experimental import pallas as pl
from jax.experimental.pallas import tpu as pltpu
from jax.experimental.pallas import tpu_sc as plsc

N = 10000
E = 320000
D = 128
DE = 4
H = 128
OUT = 10
G = 16

NC = 2
NS = 16
NW = NC * NS

C = 64
KCH = 160
EW = KCH * C
EPAD = NW * EW
N8 = N + 8
RPT = 624
RCH = 48
RTAIL = N - NS * RPT

RB = 1000
NB = N // RB


def _sc_body(table, src3, dst3, zx, out_x, src_idx, dst_idx, rows, sem, acc_x):
    c = lax.axis_index("c")
    s = lax.axis_index("s")
    wid = s * NC + c
    base = s * RPT

    pltpu.sync_copy(zx, rows)
    for k in range(RPT // RCH):
        off = base + k * RCH
        pltpu.sync_copy(rows.at[pl.ds(0, RCH)], acc_x.at[pl.ds(off, RCH)])

    @pl.when(s == NS - 1)
    def _tail_init():
        toff = NS * RPT
        pltpu.sync_copy(rows.at[pl.ds(0, RTAIL + 8)],
                        acc_x.at[pl.ds(toff, RTAIL + 8)])

    plsc.subcore_barrier()

    def chunk(t, carry):
        pltpu.sync_copy(src3.at[wid, t], src_idx)
        pltpu.sync_copy(dst3.at[wid, t], dst_idx)
        pltpu.async_copy(table.at[src_idx], rows, sem).wait()
        pltpu.sync_copy(rows, acc_x.at[dst_idx], add=True)
        return carry

    lax.fori_loop(0, KCH, chunk, 0)
    plsc.subcore_barrier()

    for k in range(RPT // RCH):
        off = base + k * RCH
        pltpu.sync_copy(acc_x.at[pl.ds(off, RCH)], rows.at[pl.ds(0, RCH)])
        pltpu.sync_copy(rows.at[pl.ds(0, RCH)],
                        out_x.at[c, pl.ds(off, RCH)])

    @pl.when(s == NS - 1)
    def _tail_out():
        toff = NS * RPT
        pltpu.sync_copy(acc_x.at[pl.ds(toff, RTAIL)],
                        rows.at[pl.ds(0, RTAIL)])
        pltpu.sync_copy(rows.at[pl.ds(0, RTAIL)],
                        out_x.at[c, pl.ds(toff, RTAIL)])


_sc_scatter = pl.kernel(
    _sc_body,
    out_type=jax.ShapeDtypeStruct((NC, N8, D), jnp.float32),
    mesh=plsc.VectorSubcoreMesh(core_axis_name="c", subcore_axis_name="s"),
    scratch_types=[
        pltpu.VMEM((C,), jnp.int32),
        pltpu.VMEM((C,), jnp.int32),
        pltpu.VMEM((C, D), jnp.float32),
        pltpu.SemaphoreType.DMA,
        pltpu.VMEM_SHARED((N8, D), jnp.float32),
    ],
)


def _tc_layer_body(S_ref, x_ref, ae_ref, Wx_ref, We_ref, b_ref, o_ref):
    aggr = S_ref[0] + S_ref[1] + x_ref[...]
    ae = ae_ref[0] + ae_ref[1]
    acc = jnp.dot(aggr, Wx_ref[...], preferred_element_type=jnp.float32)
    acc = acc + jnp.dot(ae, We_ref[...], preferred_element_type=jnp.float32)
    o_ref[...] = jnp.maximum(acc + b_ref[...], 0.0)


def _tc_final_body(S_ref, h1_ref, ae_ref, Wx_ref, We_ref, b_ref,
                   batch_ref, Wout_ref, bout_ref, o_ref, pool_acc, cnt_acc):
    i = pl.program_id(0)

    @pl.when(i == 0)
    def _init():
        pool_acc[...] = jnp.zeros_like(pool_acc)
        cnt_acc[...] = jnp.zeros_like(cnt_acc)

    aggr = S_ref[0] + S_ref[1] + h1_ref[...]
    ae = ae_ref[0] + ae_ref[1]
    h = jnp.dot(aggr, Wx_ref[...], preferred_element_type=jnp.float32)
    h = h + jnp.dot(ae, We_ref[...], preferred_element_type=jnp.float32)
    h = jnp.maximum(h + b_ref[...], 0.0)

    b_row = batch_ref[0]
    gids = lax.broadcasted_iota(jnp.int32, (G, 1), 0)
    onehot = (b_row == gids).astype(jnp.float32)
    pool_acc[...] += lax.dot_general(
        onehot, h, (((1,), (0,)), ((), ())),
        preferred_element_type=jnp.float32)
    cnt_acc[...] += lax.dot_general(
        onehot, jnp.ones((RB, H), jnp.float32), (((1,), (0,)), ((), ())),
        preferred_element_type=jnp.float32)

    @pl.when(i == NB - 1)
    def _final():
        pooled = pool_acc[...] / jnp.maximum(cnt_acc[...], 1.0)
        logits = jnp.dot(pooled, Wout_ref[...],
                         preferred_element_type=jnp.float32) + bout_ref[...]
        m = jnp.max(logits, axis=1, keepdims=True)
        z = logits - m
        lse = jnp.log(jnp.sum(jnp.exp(z), axis=1, keepdims=True))
        o_ref[...] = z - lse


_tc_layer1 = pl.pallas_call(
    _tc_layer_body,
    grid=(NB,),
    in_specs=[
        pl.BlockSpec((NC, RB, D), lambda i: (0, i, 0)),
        pl.BlockSpec((RB, D), lambda i: (i, 0)),
        pl.BlockSpec((NC, RB, D), lambda i: (0, i, 0)),
        pl.BlockSpec((D, H), lambda i: (0, 0)),
        pl.BlockSpec((D, H), lambda i: (0, 0)),
        pl.BlockSpec((1, H), lambda i: (0, 0)),
    ],
    out_specs=pl.BlockSpec((RB, H), lambda i: (i, 0)),
    out_shape=jax.ShapeDtypeStruct((N, H), jnp.float32),
)

_tc_final = pl.pallas_call(
    _tc_final_body,
    grid=(NB,),
    in_specs=[
        pl.BlockSpec((NC, RB, H), lambda i: (0, i, 0)),
        pl.BlockSpec((RB, H), lambda i: (i, 0)),
        pl.BlockSpec((NC, RB, D), lambda i: (0, i, 0)),
        pl.BlockSpec((H, H), lambda i: (0, 0)),
        pl.BlockSpec((D, H), lambda i: (0, 0)),
        pl.BlockSpec((1, H), lambda i: (0, 0)),
        pl.BlockSpec((1, 1, RB), lambda i: (i, 0, 0)),
        pl.BlockSpec((H, OUT), lambda i: (0, 0)),
        pl.BlockSpec((1, OUT), lambda i: (0, 0)),
    ],
    out_specs=pl.BlockSpec((G, OUT), lambda i: (0, 0)),
    out_shape=jax.ShapeDtypeStruct((G, OUT), jnp.float32),
    scratch_shapes=[
        pltpu.VMEM((G, H), jnp.float32),
        pltpu.VMEM((G, H), jnp.float32),
    ],
)


def kernel(x, edge_index, edge_attr, batch, W0, b0, W1, b1, Wout, bout):
    pad = EPAD - E
    src = jnp.concatenate([edge_index[0], jnp.zeros((pad,), edge_index.dtype)])
    dst = jnp.concatenate([edge_index[1], jnp.full((pad,), N, edge_index.dtype)])
    srcp = src.reshape(NW, KCH, C)
    dstp = dst.reshape(NW, KCH, C)
    iotap = jnp.arange(EPAD, dtype=jnp.int32).reshape(NW, KCH, C)
    ea128 = jnp.pad(edge_attr, ((0, pad), (0, D - DE)))
    zx = jnp.zeros((C, D), jnp.float32)

    We0 = jnp.pad(W0[D:], ((0, D - DE), (0, 0)))
    We1 = jnp.pad(W1[H:], ((0, D - DE), (0, 0)))

    Ae = _sc_scatter(ea128, iotap, dstp, zx)
    S1 = _sc_scatter(x, srcp, dstp, zx)
    h = _tc_layer1(S1, x, Ae, W0[:D], We0, b0.reshape(1, H))
    S2 = _sc_scatter(h, srcp, dstp, zx)
    return _tc_final(S2, h, Ae, W1[:H], We1, b1.reshape(1, H),
                     batch.reshape(NB, 1, RB), Wout, bout.reshape(1, OUT))


# speedup vs baseline: 4.0204x; 1.2331x over previous
import jax
import jax.numpy as jnp
from jax import lax
from jax.experimental import pallas as pl
from jax.experimental.pallas import tpu as pltpu
from jax.experimental.pallas import tpu_sc as plsc

N = 10000
E = 320000
D = 128
DE = 4
H = 128
OUT = 10
G = 16

NC = 2
NS = 16
NW = NC * NS

C = 128
KCH = 80
EW = KCH * C
EPAD = NW * EW
N8 = N + 8
RPT = 624
RCH = 48
RTAIL = N - NS * RPT

RB = 1000
NB = N // RB


def _sc_body(table, src3, dst3, zx, out_x, src_idx, dst_idx, rows, sem, acc_x):
    c = lax.axis_index("c")
    s = lax.axis_index("s")
    wid = s * NC + c
    base = s * RPT

    pltpu.sync_copy(zx, rows)
    for k in range(RPT // RCH):
        off = base + k * RCH
        pltpu.sync_copy(rows.at[pl.ds(0, RCH)], acc_x.at[pl.ds(off, RCH)])

    @pl.when(s == NS - 1)
    def _tail_init():
        toff = NS * RPT
        pltpu.sync_copy(rows.at[pl.ds(0, RTAIL + 8)],
                        acc_x.at[pl.ds(toff, RTAIL + 8)])

    plsc.subcore_barrier()

    def chunk(t, carry):
        pltpu.sync_copy(src3.at[wid, t], src_idx)
        pltpu.sync_copy(dst3.at[wid, t], dst_idx)
        pltpu.async_copy(table.at[src_idx], rows, sem).wait()
        pltpu.sync_copy(rows, acc_x.at[dst_idx], add=True)
        return carry

    lax.fori_loop(0, KCH, chunk, 0)
    plsc.subcore_barrier()

    for k in range(RPT // RCH):
        off = base + k * RCH
        pltpu.sync_copy(acc_x.at[pl.ds(off, RCH)], rows.at[pl.ds(0, RCH)])
        pltpu.sync_copy(rows.at[pl.ds(0, RCH)],
                        out_x.at[c, pl.ds(off, RCH)])

    @pl.when(s == NS - 1)
    def _tail_out():
        toff = NS * RPT
        pltpu.sync_copy(acc_x.at[pl.ds(toff, RTAIL)],
                        rows.at[pl.ds(0, RTAIL)])
        pltpu.sync_copy(rows.at[pl.ds(0, RTAIL)],
                        out_x.at[c, pl.ds(toff, RTAIL)])


_sc_scatter = pl.kernel(
    _sc_body,
    out_type=jax.ShapeDtypeStruct((NC, N8, D), jnp.float32),
    mesh=plsc.VectorSubcoreMesh(core_axis_name="c", subcore_axis_name="s"),
    scratch_types=[
        pltpu.VMEM((C,), jnp.int32),
        pltpu.VMEM((C,), jnp.int32),
        pltpu.VMEM((C, D), jnp.float32),
        pltpu.SemaphoreType.DMA,
        pltpu.VMEM_SHARED((N8, D), jnp.float32),
    ],
)


def _tc_layer_body(S_ref, x_ref, ae_ref, Wx_ref, We_ref, b_ref, o_ref):
    aggr = S_ref[0] + S_ref[1] + x_ref[...]
    ae = ae_ref[0] + ae_ref[1]
    acc = jnp.dot(aggr, Wx_ref[...], preferred_element_type=jnp.float32)
    acc = acc + jnp.dot(ae, We_ref[...], preferred_element_type=jnp.float32)
    o_ref[...] = jnp.maximum(acc + b_ref[...], 0.0)


def _tc_final_body(S_ref, h1_ref, ae_ref, Wx_ref, We_ref, b_ref,
                   batch_ref, Wout_ref, bout_ref, o_ref, pool_acc, cnt_acc):
    i = pl.program_id(0)

    @pl.when(i == 0)
    def _init():
        pool_acc[...] = jnp.zeros_like(pool_acc)
        cnt_acc[...] = jnp.zeros_like(cnt_acc)

    aggr = S_ref[0] + S_ref[1] + h1_ref[...]
    ae = ae_ref[0] + ae_ref[1]
    h = jnp.dot(aggr, Wx_ref[...], preferred_element_type=jnp.float32)
    h = h + jnp.dot(ae, We_ref[...], preferred_element_type=jnp.float32)
    h = jnp.maximum(h + b_ref[...], 0.0)

    b_row = batch_ref[0]
    gids = lax.broadcasted_iota(jnp.int32, (G, 1), 0)
    onehot = (b_row == gids).astype(jnp.float32)
    pool_acc[...] += lax.dot_general(
        onehot, h, (((1,), (0,)), ((), ())),
        preferred_element_type=jnp.float32)
    cnt_acc[...] += lax.dot_general(
        onehot, jnp.ones((RB, H), jnp.float32), (((1,), (0,)), ((), ())),
        preferred_element_type=jnp.float32)

    @pl.when(i == NB - 1)
    def _final():
        pooled = pool_acc[...] / jnp.maximum(cnt_acc[...], 1.0)
        logits = jnp.dot(pooled, Wout_ref[...],
                         preferred_element_type=jnp.float32) + bout_ref[...]
        m = jnp.max(logits, axis=1, keepdims=True)
        z = logits - m
        lse = jnp.log(jnp.sum(jnp.exp(z), axis=1, keepdims=True))
        o_ref[...] = z - lse


_tc_layer1 = pl.pallas_call(
    _tc_layer_body,
    grid=(NB,),
    in_specs=[
        pl.BlockSpec((NC, RB, D), lambda i: (0, i, 0)),
        pl.BlockSpec((RB, D), lambda i: (i, 0)),
        pl.BlockSpec((NC, RB, D), lambda i: (0, i, 0)),
        pl.BlockSpec((D, H), lambda i: (0, 0)),
        pl.BlockSpec((D, H), lambda i: (0, 0)),
        pl.BlockSpec((1, H), lambda i: (0, 0)),
    ],
    out_specs=pl.BlockSpec((RB, H), lambda i: (i, 0)),
    out_shape=jax.ShapeDtypeStruct((N, H), jnp.float32),
)

_tc_final = pl.pallas_call(
    _tc_final_body,
    grid=(NB,),
    in_specs=[
        pl.BlockSpec((NC, RB, H), lambda i: (0, i, 0)),
        pl.BlockSpec((RB, H), lambda i: (i, 0)),
        pl.BlockSpec((NC, RB, D), lambda i: (0, i, 0)),
        pl.BlockSpec((H, H), lambda i: (0, 0)),
        pl.BlockSpec((D, H), lambda i: (0, 0)),
        pl.BlockSpec((1, H), lambda i: (0, 0)),
        pl.BlockSpec((1, 1, RB), lambda i: (i, 0, 0)),
        pl.BlockSpec((H, OUT), lambda i: (0, 0)),
        pl.BlockSpec((1, OUT), lambda i: (0, 0)),
    ],
    out_specs=pl.BlockSpec((G, OUT), lambda i: (0, 0)),
    out_shape=jax.ShapeDtypeStruct((G, OUT), jnp.float32),
    scratch_shapes=[
        pltpu.VMEM((G, H), jnp.float32),
        pltpu.VMEM((G, H), jnp.float32),
    ],
)


def kernel(x, edge_index, edge_attr, batch, W0, b0, W1, b1, Wout, bout):
    pad = EPAD - E
    src = jnp.concatenate([edge_index[0], jnp.zeros((pad,), edge_index.dtype)])
    dst = jnp.concatenate([edge_index[1], jnp.full((pad,), N, edge_index.dtype)])
    srcp = src.reshape(NW, KCH, C)
    dstp = dst.reshape(NW, KCH, C)
    iotap = jnp.arange(EPAD, dtype=jnp.int32).reshape(NW, KCH, C)
    ea128 = jnp.pad(edge_attr, ((0, pad), (0, D - DE)))
    zx = jnp.zeros((C, D), jnp.float32)

    We0 = jnp.pad(W0[D:], ((0, D - DE), (0, 0)))
    We1 = jnp.pad(W1[H:], ((0, D - DE), (0, 0)))

    Ae = _sc_scatter(ea128, iotap, dstp, zx)
    S1 = _sc_scatter(x, srcp, dstp, zx)
    h = _tc_layer1(S1, x, Ae, W0[:D], We0, b0.reshape(1, H))
    S2 = _sc_scatter(h, srcp, dstp, zx)
    return _tc_final(S2, h, Ae, W1[:H], We1, b1.reshape(1, H),
                     batch.reshape(NB, 1, RB), Wout, bout.reshape(1, OUT))
